# Initial kernel scaffold; baseline (speedup 1.0000x reference)
import jax
import jax.numpy as jnp
from jax.experimental import pallas as pl


def kernel(data, org_edge_index, emb, lin_W, att_i, att_j, att_em_i, att_em_j, gl_bias, bn1_g, bn1_b, bn2_g, bn2_b, out_W, out_b):
    raise NotImplementedError("write your pallas kernel here")


# speedup vs baseline: 13.4064x; 13.4064x over previous
import functools

import jax
import jax.numpy as jnp
from jax import lax
from jax.experimental import pallas as pl
from jax.experimental.pallas import tpu as pltpu
from jax.experimental.pallas import tpu_sc as plsc

BATCH = 4
NODE_NUM = 10000
INPUT_DIM = 10
DIM = 64
TOPK = 20

N_TOTAL = BATCH * NODE_NUM
NEGF = -1e30
BIGI = 2**30


CA = 128
GRID_TK = (NODE_NUM + CA - 1) // CA
NPAD = GRID_TK * CA


def _topk_body(emb_ref, embT_ref, tkT_ref, s_ref):
    emb = emb_ref[...]
    nrm2 = jnp.sum(emb * emb, axis=1, keepdims=True)
    embT = embT_ref[...]
    nrm2T = jnp.sum(embT * embT, axis=0, keepdims=True)
    s0 = jnp.dot(emb, embT,
                 preferred_element_type=jnp.float32)
    s0 = s0 / (jnp.sqrt(nrm2) * jnp.sqrt(nrm2T))
    s_ref[...] = s0
    ii = lax.broadcasted_iota(jnp.int32, (NODE_NUM, CA), 0)

    def step(k, m):
        s = s_ref[...]
        em = s == m
        idx = jnp.min(jnp.where(em, ii, BIGI), axis=0, keepdims=True)
        sn = jnp.where(em, NEGF, s)
        s_ref[...] = sn
        tkT_ref[pl.ds(k, 1), :] = idx
        return jnp.max(sn, axis=0, keepdims=True)

    lax.fori_loop(0, TOPK, step, jnp.max(s0, axis=0, keepdims=True))


def _run_topk(emb, embT):
    return pl.pallas_call(
        _topk_body,
        grid=(GRID_TK,),
        in_specs=[
            pl.BlockSpec((NODE_NUM, DIM), lambda i: (0, 0)),
            pl.BlockSpec((DIM, CA), lambda i: (0, i)),
        ],
        out_specs=pl.BlockSpec((TOPK, CA), lambda i: (0, i)),
        out_shape=jax.ShapeDtypeStruct((TOPK, NPAD), jnp.int32),
        scratch_shapes=[pltpu.VMEM((NODE_NUM, CA), jnp.float32)],
    )(emb, embT)


GRID_X = 50
RX = N_TOTAL // GRID_X


def _lin_body(x_ref, emb4_ref, linW_ref, ai_ref, aj_ref, aei_ref, aej_ref,
              xl_ref, ci_ref, cj_ref):
    xl = jnp.dot(x_ref[...], linW_ref[...],
                 preferred_element_type=jnp.float32)
    xl_ref[...] = xl
    e4 = emb4_ref[...]
    ci_ref[...] = (jnp.sum(xl * ai_ref[...], axis=1, keepdims=True)
                   + jnp.sum(e4 * aei_ref[...], axis=1, keepdims=True))
    cj_ref[...] = (jnp.sum(xl * aj_ref[...], axis=1, keepdims=True)
                   + jnp.sum(e4 * aej_ref[...], axis=1, keepdims=True))


def _run_lin(x, emb4, lin_W, ai, aj, aei, aej):
    full = lambda i: (0, 0)
    row = lambda i: (i, 0)
    return pl.pallas_call(
        _lin_body,
        grid=(GRID_X,),
        in_specs=[
            pl.BlockSpec((RX, INPUT_DIM), row),
            pl.BlockSpec((RX, DIM), row),
            pl.BlockSpec((INPUT_DIM, DIM), full),
            pl.BlockSpec((1, DIM), full),
            pl.BlockSpec((1, DIM), full),
            pl.BlockSpec((1, DIM), full),
            pl.BlockSpec((1, DIM), full),
        ],
        out_specs=[
            pl.BlockSpec((RX, DIM), row),
            pl.BlockSpec((RX, 1), row),
            pl.BlockSpec((RX, 1), row),
        ],
        out_shape=[
            jax.ShapeDtypeStruct((N_TOTAL, DIM), jnp.float32),
            jax.ShapeDtypeStruct((N_TOTAL, 1), jnp.float32),
            jax.ShapeDtypeStruct((N_TOTAL, 1), jnp.float32),
        ],
    )(x, emb4, lin_W, ai, aj, aei, aej)


NW = 32
NPW = N_TOTAL // NW
CHUNK = 5
KSLOT = 24
NCHUNK = NPW // CHUNK


def _stage_b_body(xl_hbm, ci_hbm, cj_hbm, nbr_hbm, out_hbm,
                  cib, cjb, idxs, rows, outc, sem):
    wid = lax.axis_index("s") * 2 + lax.axis_index("c")
    base = wid * NPW
    bb = (base // NODE_NUM) * NODE_NUM

    pltpu.sync_copy(ci_hbm.at[pl.ds(bb, NODE_NUM)], cib)
    pltpu.sync_copy(cj_hbm.at[pl.ds(bb, NODE_NUM)], cjb)

    lane = lax.broadcasted_iota(jnp.int32, (16,), 0)

    def chunk_body(c, carry):
        nb = base + c * CHUNK
        pltpu.sync_copy(nbr_hbm.at[pl.ds(nb * KSLOT, CHUNK * KSLOT)], idxs)
        pltpu.async_copy(xl_hbm.at[idxs], rows, sem).wait()
        for i in range(CHUNK):
            n = nb + i
            off = i * KSLOT
            iv0 = idxs[pl.ds(off, 16)]
            iv1 = idxs[pl.ds(off + 8, 16)]
            cj0 = plsc.load_gather(cjb, [iv0 - bb])
            cj1 = plsc.load_gather(cjb, [iv1 - bb])
            ci_s = plsc.load_gather(cib, [jnp.full((16,), n - bb, jnp.int32)])
            z0 = ci_s + cj0
            z1 = ci_s + cj1
            a0 = jnp.where(z0 > 0, z0, jnp.float32(0.2) * z0)
            a1 = jnp.where(z1 > 0, z1, jnp.float32(0.2) * z1)
            nsp = jnp.full((16,), n, jnp.int32)
            p1 = lane + 8
            act0 = iv0 != nsp
            act1 = (p1 == TOPK) | ((p1 >= 16) & (p1 < TOPK) & (iv1 != nsp))
            m0 = jnp.where(act0, a0, NEGF)
            m1 = jnp.where(act1, a1, NEGF)
            mx = jnp.maximum(jnp.max(m0, axis=0), jnp.max(m1, axis=0))
            e0 = jnp.exp(m0 - mx)
            e1 = jnp.exp(m1 - mx)
            ssum = jnp.sum(e0, axis=0) + jnp.sum(e1, axis=0)
            dv = jnp.broadcast_to(ssum + jnp.float32(1e-16), (16,))
            ek = [jnp.max(jnp.where(lane == k, e0, 0.0), axis=0)
                  for k in range(16)]
            ek += [jnp.max(jnp.where(lane == (k - 8), e1, 0.0), axis=0)
                   for k in range(16, TOPK + 1)]
            for j in range(DIM // 16):
                acc = jnp.zeros((16,), jnp.float32)
                for k in range(TOPK + 1):
                    acc = acc + ek[k] * rows[off + k, pl.ds(j * 16, 16)]
                outc[pl.ds(i * DIM + j * 16, 16)] = acc / dv
        pltpu.sync_copy(outc, out_hbm.at[pl.ds(nb * DIM, CHUNK * DIM)])
        return carry

    lax.fori_loop(0, NCHUNK, chunk_body, 0)


def _run_stage_b(xl, ci, cj, nbr):
    mesh = plsc.VectorSubcoreMesh(core_axis_name="c", subcore_axis_name="s")
    f = functools.partial(
        pl.kernel, mesh=mesh,
        compiler_params=pltpu.CompilerParams(needs_layout_passes=False,
                                             use_tc_tiling_on_sc=False),
        out_type=jax.ShapeDtypeStruct((N_TOTAL * DIM,), jnp.float32),
        scratch_types=[
            pltpu.VMEM((NODE_NUM,), jnp.float32),
            pltpu.VMEM((NODE_NUM,), jnp.float32),
            pltpu.VMEM((CHUNK * KSLOT,), jnp.int32),
            pltpu.VMEM((CHUNK * KSLOT, DIM), jnp.float32),
            pltpu.VMEM((CHUNK * DIM,), jnp.float32),
            pltpu.SemaphoreType.DMA,
        ],
    )(_stage_b_body)
    return f(xl, ci, cj, nbr)


def _sums_body(h_ref, s1_ref, s2_ref):
    @pl.when(pl.program_id(0) == 0)
    def _():
        s1_ref[...] = jnp.zeros_like(s1_ref)
        s2_ref[...] = jnp.zeros_like(s2_ref)
    h = h_ref[...]
    s1_ref[...] += jnp.sum(h, axis=0, keepdims=True)
    s2_ref[...] += jnp.sum(h * h, axis=0, keepdims=True)


def _run_sums(h):
    return pl.pallas_call(
        _sums_body,
        grid=(GRID_X,),
        in_specs=[pl.BlockSpec((RX, DIM), lambda i: (i, 0))],
        out_specs=[pl.BlockSpec((1, DIM), lambda i: (0, 0)),
                   pl.BlockSpec((1, DIM), lambda i: (0, 0))],
        out_shape=[jax.ShapeDtypeStruct((1, DIM), jnp.float32),
                   jax.ShapeDtypeStruct((1, DIM), jnp.float32)],
    )(h)


def _bn1_body(h_ref, emb4_ref, mu_ref, rs_ref, g_ref, b_ref,
              h1_ref, s1_ref, s2_ref):
    @pl.when(pl.program_id(0) == 0)
    def _():
        s1_ref[...] = jnp.zeros_like(s1_ref)
        s2_ref[...] = jnp.zeros_like(s2_ref)
    h = (h_ref[...] - mu_ref[...]) * rs_ref[...] * g_ref[...] + b_ref[...]
    h = jnp.maximum(h, 0.0) * emb4_ref[...]
    h1_ref[...] = h
    s1_ref[...] += jnp.sum(h, axis=0, keepdims=True)
    s2_ref[...] += jnp.sum(h * h, axis=0, keepdims=True)


def _run_bn1(h, emb4, mu, rs, g, b):
    full = lambda i: (0, 0)
    row = lambda i: (i, 0)
    return pl.pallas_call(
        _bn1_body,
        grid=(GRID_X,),
        in_specs=[pl.BlockSpec((RX, DIM), row), pl.BlockSpec((RX, DIM), row),
                  pl.BlockSpec((1, DIM), full), pl.BlockSpec((1, DIM), full),
                  pl.BlockSpec((1, DIM), full), pl.BlockSpec((1, DIM), full)],
        out_specs=[pl.BlockSpec((RX, DIM), row),
                   pl.BlockSpec((1, DIM), full), pl.BlockSpec((1, DIM), full)],
        out_shape=[jax.ShapeDtypeStruct((N_TOTAL, DIM), jnp.float32),
                   jax.ShapeDtypeStruct((1, DIM), jnp.float32),
                   jax.ShapeDtypeStruct((1, DIM), jnp.float32)],
    )(h, emb4, mu, rs, g, b)


def _bn2_body(h_ref, mu_ref, rs_ref, g_ref, b_ref, ow_ref, ob_ref, y_ref):
    h = (h_ref[...] - mu_ref[...]) * rs_ref[...] * g_ref[...] + b_ref[...]
    h = jnp.maximum(h, 0.0)
    y_ref[...] = jnp.sum(h * ow_ref[...], axis=1, keepdims=True) + ob_ref[...]


def _run_bn2(h, mu, rs, g, b, ow, ob):
    full = lambda i: (0, 0)
    row = lambda i: (i, 0)
    return pl.pallas_call(
        _bn2_body,
        grid=(GRID_X,),
        in_specs=[pl.BlockSpec((RX, DIM), row),
                  pl.BlockSpec((1, DIM), full), pl.BlockSpec((1, DIM), full),
                  pl.BlockSpec((1, DIM), full), pl.BlockSpec((1, DIM), full),
                  pl.BlockSpec((1, DIM), full),
                  pl.BlockSpec((1, 1), full)],
        out_specs=pl.BlockSpec((RX, 1), row),
        out_shape=jax.ShapeDtypeStruct((N_TOTAL, 1), jnp.float32),
    )(h, mu, rs, g, b, ow, ob)


def kernel(data, org_edge_index, emb, lin_W, att_i, att_j, att_em_i,
           att_em_j, gl_bias, bn1_g, bn1_b, bn2_g, bn2_b, out_W, out_b):
    del org_edge_index, gl_bias
    x = data.reshape(-1, INPUT_DIM)
    embT = emb.T
    emb4 = jnp.tile(emb, (BATCH, 1))
    r1 = lambda v: v.reshape(1, -1)

    tkT = _run_topk(emb, embT)
    topk = tkT[:, :NODE_NUM].T
    xl, ci, cj = _run_lin(x, emb4, lin_W, r1(att_i), r1(att_j),
                          r1(att_em_i), r1(att_em_j))

    selfc = jnp.broadcast_to(jnp.arange(NODE_NUM, dtype=jnp.int32)[:, None],
                             (NODE_NUM, KSLOT - TOPK))
    nbr = jnp.concatenate([topk, selfc], axis=1)
    offs = (jnp.arange(BATCH, dtype=jnp.int32) * NODE_NUM)[:, None, None]
    nbr = (nbr[None] + offs).reshape(-1)

    outb = _run_stage_b(xl, ci.reshape(-1), cj.reshape(-1), nbr)
    outb = outb.reshape(N_TOTAL, DIM)

    inv_n = 1.0 / N_TOTAL
    s1, s2 = _run_sums(outb)
    mu1 = s1 * inv_n
    rs1 = 1.0 / jnp.sqrt(s2 * inv_n - mu1 * mu1 + 1e-5)
    h1, t1, t2 = _run_bn1(outb, emb4, mu1, rs1, r1(bn1_g), r1(bn1_b))
    mu2 = t1 * inv_n
    rs2 = 1.0 / jnp.sqrt(t2 * inv_n - mu2 * mu2 + 1e-5)
    y = _run_bn2(h1, mu2, rs2, r1(bn2_g), r1(bn2_b), out_W.reshape(1, -1),
                 out_b.reshape(1, 1))
    return y.reshape(BATCH, NODE_NUM)
